# Initial kernel scaffold; baseline (speedup 1.0000x reference)
import jax
import jax.numpy as jnp
from jax.experimental import pallas as pl


def kernel(x, edge_index, W1, att_src1, att_dst1, bias1, W2, att_src2, att_dst2, bias2):
    raise NotImplementedError("write your pallas kernel here")


# speedup vs baseline: 11.1529x; 11.1529x over previous
import jax
import jax.numpy as jnp
from jax import lax
from jax.experimental import pallas as pl
from jax.experimental.pallas import tpu as pltpu
from jax.experimental.pallas import tpu_sc as plsc

_FW = 64
_RT = 400
_C = 80
_NCORE = 2
_NSUB = 16


def _leaky(x):
    return jnp.where(x >= 0, x, 0.2 * x)


def _prep1_body(x_ref, w1_ref, as_ref, ad_ref, xw8_ref, asrc_ref, adst_ref):
    xw = jnp.dot(x_ref[...], w1_ref[...], preferred_element_type=jnp.float32)
    asrc_ref[...] = jnp.dot(xw, as_ref[...], preferred_element_type=jnp.float32)
    adst_ref[...] = jnp.dot(xw, ad_ref[...], preferred_element_type=jnp.float32)
    for p in range(xw8_ref.shape[0]):
        xw8_ref[p] = xw[:, _FW * p:_FW * (p + 1)]


def _prep1(x, W1, As, Ad, N, K, HID):
    P = (K * HID) // _FW
    G = N // _RT
    return pl.pallas_call(
        _prep1_body,
        grid=(G,),
        in_specs=[
            pl.BlockSpec((_RT, x.shape[1]), lambda i: (i, 0)),
            pl.BlockSpec(W1.shape, lambda i: (0, 0)),
            pl.BlockSpec(As.shape, lambda i: (0, 0)),
            pl.BlockSpec(Ad.shape, lambda i: (0, 0)),
        ],
        out_specs=[
            pl.BlockSpec((P, _RT, _FW), lambda i: (0, i, 0)),
            pl.BlockSpec((_RT, K), lambda i: (i, 0)),
            pl.BlockSpec((_RT, K), lambda i: (i, 0)),
        ],
        out_shape=[
            jax.ShapeDtypeStruct((P, N, _FW), jnp.float32),
            jax.ShapeDtypeStruct((N, K), jnp.float32),
            jax.ShapeDtypeStruct((N, K), jnp.float32),
        ],
    )(x, W1, As, Ad)


def _sc_edges(src, dst, xwp, aS, aD, N, E, P, hmap):
    epw = E // (_NCORE * _NSUB)
    nchunk = epw // _C
    rps = N // _NSUB
    ngrp = _C // 16
    zfull, zrem = rps // _C, rps % _C

    mesh = plsc.VectorSubcoreMesh(core_axis_name="c", subcore_axis_name="s")

    def body(src_h, dst_h, xwp_h, aS_h, aD_h, acc_o, dnm_o,
             acc_sp, dnm_sp, tS, tD, srcb, dstb, sidxb, rows, wbuf, w0b, sem):
        cc = lax.axis_index("c")
        ss = lax.axis_index("s")
        wid = cc * _NSUB + ss
        lanes = lax.iota(jnp.int32, 16)
        zcol = jnp.zeros((16,), jnp.int32)
        zvec = jnp.zeros((16,), jnp.float32)

        def zr(e, _):
            for j in range(_FW // 16):
                rows[e, pl.ds(j * 16, 16)] = zvec
            wbuf[e, pl.ds(0, 16)] = zvec
            return 0
        lax.fori_loop(0, _C, zr, 0)

        for p in range(P):
            if p > 0:
                for g in range(ngrp):
                    plsc.store_scatter(wbuf, [g * 16 + lanes, zcol], zvec)
            for z in range(zfull):
                pltpu.sync_copy(rows, acc_sp.at[pl.ds(ss * rps + z * _C, _C)])
                pltpu.sync_copy(wbuf, dnm_sp.at[pl.ds(ss * rps + z * _C, _C)])
            if zrem:
                pltpu.sync_copy(rows.at[pl.ds(0, zrem)],
                                acc_sp.at[pl.ds(ss * rps + zfull * _C, zrem)])
                pltpu.sync_copy(wbuf.at[pl.ds(0, zrem)],
                                dnm_sp.at[pl.ds(ss * rps + zfull * _C, zrem)])
            if p == 0 or hmap[p] != hmap[p - 1]:
                pltpu.sync_copy(aS_h.at[pl.ds(hmap[p] * N, N)], tS)
                pltpu.sync_copy(aD_h.at[pl.ds(hmap[p] * N, N)], tD)
            plsc.subcore_barrier()

            def chunk(ci, _):
                eb = wid * epw + ci * _C
                pltpu.sync_copy(src_h.at[pl.ds(eb, _C)], srcb)
                pltpu.sync_copy(dst_h.at[pl.ds(eb, _C)], dstb)

                def grp(g, _):
                    sl = pl.ds(g * 16, 16)
                    sv = srcb[sl]
                    dv = dstb[sl]
                    sidxb[sl] = sv + p * N
                    a1 = plsc.load_gather(tS, [sv])
                    a2 = plsc.load_gather(tD, [dv])
                    w = jnp.exp(_leaky(a1 + a2))
                    w0b[sl] = w
                    plsc.store_scatter(wbuf, [g * 16 + lanes, zcol], w)
                    return 0
                lax.fori_loop(0, ngrp, grp, 0)

                pltpu.async_copy(xwp_h.at[sidxb], rows, sem).wait()

                def mul(g, _):
                    base = g * 16
                    wv = w0b[pl.ds(base, 16)]
                    for i in range(16):
                        e = base + i
                        w = wv[i]
                        for j in range(_FW // 16):
                            sl = pl.ds(j * 16, 16)
                            rows[e, sl] = rows[e, sl] * w
                    return 0
                lax.fori_loop(0, ngrp, mul, 0)

                pltpu.sync_copy(rows, acc_sp.at[dstb], add=True)
                pltpu.sync_copy(wbuf, dnm_sp.at[dstb], add=True)
                return 0
            lax.fori_loop(0, nchunk, chunk, 0)
            plsc.subcore_barrier()

            ob = (cc * P + p) * N + ss * rps
            for z in range(zfull):
                pltpu.sync_copy(acc_sp.at[pl.ds(ss * rps + z * _C, _C)],
                                acc_o.at[pl.ds(ob + z * _C, _C)])
                pltpu.sync_copy(dnm_sp.at[pl.ds(ss * rps + z * _C, _C)],
                                dnm_o.at[pl.ds(ob + z * _C, _C)])
            if zrem:
                pltpu.sync_copy(acc_sp.at[pl.ds(ss * rps + zfull * _C, zrem)],
                                acc_o.at[pl.ds(ob + zfull * _C, zrem)])
                pltpu.sync_copy(dnm_sp.at[pl.ds(ss * rps + zfull * _C, zrem)],
                                dnm_o.at[pl.ds(ob + zfull * _C, zrem)])
            plsc.subcore_barrier()

    k = pl.kernel(
        body,
        out_type=(jax.ShapeDtypeStruct((_NCORE * P * N, _FW), jnp.float32),
                  jax.ShapeDtypeStruct((_NCORE * P * N, 16), jnp.float32)),
        mesh=mesh,
        compiler_params=pltpu.CompilerParams(needs_layout_passes=False,
                                             use_tc_tiling_on_sc=False),
        scratch_types=[
            pltpu.VMEM_SHARED((N, _FW), jnp.float32),
            pltpu.VMEM_SHARED((N, 16), jnp.float32),
            pltpu.VMEM((N,), jnp.float32),
            pltpu.VMEM((N,), jnp.float32),
            pltpu.VMEM((_C,), jnp.int32),
            pltpu.VMEM((_C,), jnp.int32),
            pltpu.VMEM((_C,), jnp.int32),
            pltpu.VMEM((_C, _FW), jnp.float32),
            pltpu.VMEM((_C, 16), jnp.float32),
            pltpu.VMEM((_C,), jnp.float32),
            pltpu.SemaphoreType.DMA,
        ],
    )
    return k(src, dst, xwp, aS, aD)


def _epi1_body(acc_ref, dnm_ref, asrc_ref, adst_ref, xw8_ref, b1_ref, w2_ref,
               as2_ref, ad2_ref, rmat_ref, xw2_ref, a2s_ref, a2d_ref):
    P = xw8_ref.shape[0]
    numer = jnp.concatenate(
        [acc_ref[0, p] + acc_ref[1, p] for p in range(P)], axis=1)
    dnm = jnp.concatenate(
        [dnm_ref[0, p][:, 0:1] + dnm_ref[1, p][:, 0:1] for p in range(P)],
        axis=1)
    xw = jnp.concatenate([xw8_ref[p] for p in range(P)], axis=1)
    selfw = jnp.exp(_leaky(asrc_ref[...] + adst_ref[...]))
    dnm = dnm + selfw + jnp.float32(1e-16)
    rmat = rmat_ref[...]
    numer = numer + xw * jnp.dot(selfw, rmat, preferred_element_type=jnp.float32)
    h = numer / jnp.dot(dnm, rmat, preferred_element_type=jnp.float32)
    h = h + b1_ref[...]
    h = jnp.where(h > 0, h, jnp.exp(jnp.minimum(h, 0.0)) - 1.0)
    xw2 = jnp.dot(h, w2_ref[...], preferred_element_type=jnp.float32)
    for q in range(xw2_ref.shape[0]):
        xw2_ref[q] = xw2[:, _FW * q:_FW * (q + 1)]
    a2s_ref[...] = jnp.sum(xw2 * as2_ref[...], axis=1, keepdims=True)
    a2d_ref[...] = jnp.sum(xw2 * ad2_ref[...], axis=1, keepdims=True)


def _epi1(acc, dnm, a_src, a_dst, xw8, bias1, W2, as2, ad2, rmat, N, K):
    P = xw8.shape[0]
    G = N // _RT
    Q = W2.shape[1] // _FW
    return pl.pallas_call(
        _epi1_body,
        grid=(G,),
        in_specs=[
            pl.BlockSpec((2, P, _RT, _FW), lambda i: (0, 0, i, 0)),
            pl.BlockSpec((2, P, _RT, 16), lambda i: (0, 0, i, 0)),
            pl.BlockSpec((_RT, K), lambda i: (i, 0)),
            pl.BlockSpec((_RT, K), lambda i: (i, 0)),
            pl.BlockSpec((P, _RT, _FW), lambda i: (0, i, 0)),
            pl.BlockSpec((1, bias1.shape[1]), lambda i: (0, 0)),
            pl.BlockSpec(W2.shape, lambda i: (0, 0)),
            pl.BlockSpec((1, 128), lambda i: (0, 0)),
            pl.BlockSpec((1, 128), lambda i: (0, 0)),
            pl.BlockSpec(rmat.shape, lambda i: (0, 0)),
        ],
        out_specs=[
            pl.BlockSpec((Q, _RT, _FW), lambda i: (0, i, 0)),
            pl.BlockSpec((_RT, 1), lambda i: (i, 0)),
            pl.BlockSpec((_RT, 1), lambda i: (i, 0)),
        ],
        out_shape=[
            jax.ShapeDtypeStruct((Q, N, _FW), jnp.float32),
            jax.ShapeDtypeStruct((N, 1), jnp.float32),
            jax.ShapeDtypeStruct((N, 1), jnp.float32),
        ],
    )(acc, dnm, a_src, a_dst, xw8, bias1, W2, as2, ad2, rmat)


def _epi2_body(acc_ref, dnm_ref, a2s_ref, a2d_ref, xw2_ref, b2_ref, o_ref):
    Q = xw2_ref.shape[0]
    selfw = jnp.exp(_leaky(a2s_ref[...] + a2d_ref[...]))
    numer = jnp.concatenate(
        [acc_ref[0, q] + acc_ref[1, q] for q in range(Q)], axis=1)
    xw2 = jnp.concatenate([xw2_ref[q] for q in range(Q)], axis=1)
    dnm = dnm_ref[0, 0][:, 0:1] + dnm_ref[1, 0][:, 0:1] + selfw + jnp.float32(1e-16)
    o = (numer + xw2 * selfw) / dnm + b2_ref[...]
    m = jnp.max(o, axis=1, keepdims=True)
    o_ref[...] = o - (m + jnp.log(jnp.sum(jnp.exp(o - m), axis=1, keepdims=True)))


def _epi2(acc2, dnm2, a2s, a2d, xw2h, bias2, N):
    G = N // _RT
    Q = xw2h.shape[0]
    return pl.pallas_call(
        _epi2_body,
        grid=(G,),
        in_specs=[
            pl.BlockSpec((2, Q, _RT, _FW), lambda i: (0, 0, i, 0)),
            pl.BlockSpec((2, Q, _RT, 16), lambda i: (0, 0, i, 0)),
            pl.BlockSpec((_RT, 1), lambda i: (i, 0)),
            pl.BlockSpec((_RT, 1), lambda i: (i, 0)),
            pl.BlockSpec((Q, _RT, _FW), lambda i: (0, i, 0)),
            pl.BlockSpec((1, 128), lambda i: (0, 0)),
        ],
        out_specs=pl.BlockSpec((_RT, 128), lambda i: (i, 0)),
        out_shape=jax.ShapeDtypeStruct((N, 128), jnp.float32),
    )(acc2, dnm2, a2s, a2d, xw2h, bias2)


def kernel(x, edge_index, W1, att_src1, att_dst1, bias1,
           W2, att_src2, att_dst2, bias2):
    N, D = x.shape
    E = edge_index.shape[1]
    K, HID = att_src1.shape[1], att_src1.shape[2]
    P1 = (K * HID) // _FW
    Q = W2.shape[1] // _FW

    src = edge_index[0]
    dst = edge_index[1]

    eye = jnp.eye(K, dtype=jnp.float32)
    As1 = (att_src1[0][:, :, None] * eye[:, None, :]).reshape(K * HID, K)
    Ad1 = (att_dst1[0][:, :, None] * eye[:, None, :]).reshape(K * HID, K)
    rmat = jnp.repeat(eye, HID, axis=1)

    xw8, a_src, a_dst = _prep1(x, W1, As1, Ad1, N, K, HID)
    aS1 = a_src.T.reshape(-1)
    aD1 = a_dst.T.reshape(-1)
    acc1, dnm1 = _sc_edges(src, dst, xw8.reshape(P1 * N, _FW), aS1, aD1,
                           N, E, P1, list(range(P1)))

    xw2h, a2s, a2d = _epi1(acc1.reshape(2, P1, N, _FW),
                           dnm1.reshape(2, P1, N, 16), a_src, a_dst, xw8,
                           bias1.reshape(1, -1), W2,
                           att_src2.reshape(1, 128), att_dst2.reshape(1, 128),
                           rmat, N, K)
    acc2, dnm2 = _sc_edges(src, dst, xw2h.reshape(Q * N, _FW),
                           a2s.reshape(-1), a2d.reshape(-1), N, E, Q, [0] * Q)

    return _epi2(acc2.reshape(2, Q, N, _FW), dnm2.reshape(2, Q, N, 16),
                 a2s, a2d, xw2h, bias2.reshape(1, 128), N)


# speedup vs baseline: 14.2455x; 1.2773x over previous
import jax
import jax.numpy as jnp
from jax import lax
from jax.experimental import pallas as pl
from jax.experimental.pallas import tpu as pltpu
from jax.experimental.pallas import tpu_sc as plsc

_FW = 64
_RT = 400
_C = 80
_NCORE = 2
_NSUB = 16


def _leaky(x):
    return jnp.where(x >= 0, x, 0.2 * x)


def _prep1_body(x_ref, w1_ref, as_ref, ad_ref, xw8_ref, asrc_ref, adst_ref):
    xw = jnp.dot(x_ref[...], w1_ref[...], preferred_element_type=jnp.float32)
    asrc_ref[...] = jnp.dot(xw, as_ref[...], preferred_element_type=jnp.float32)
    adst_ref[...] = jnp.dot(xw, ad_ref[...], preferred_element_type=jnp.float32)
    for p in range(xw8_ref.shape[0]):
        xw8_ref[p] = xw[:, _FW * p:_FW * (p + 1)]


def _prep1(x, W1, As, Ad, N, K, HID):
    P = (K * HID) // _FW
    G = N // _RT
    return pl.pallas_call(
        _prep1_body,
        grid=(G,),
        in_specs=[
            pl.BlockSpec((_RT, x.shape[1]), lambda i: (i, 0)),
            pl.BlockSpec(W1.shape, lambda i: (0, 0)),
            pl.BlockSpec(As.shape, lambda i: (0, 0)),
            pl.BlockSpec(Ad.shape, lambda i: (0, 0)),
        ],
        out_specs=[
            pl.BlockSpec((P, _RT, _FW), lambda i: (0, i, 0)),
            pl.BlockSpec((_RT, K), lambda i: (i, 0)),
            pl.BlockSpec((_RT, K), lambda i: (i, 0)),
        ],
        out_shape=[
            jax.ShapeDtypeStruct((P, N, _FW), jnp.float32),
            jax.ShapeDtypeStruct((N, K), jnp.float32),
            jax.ShapeDtypeStruct((N, K), jnp.float32),
        ],
    )(x, W1, As, Ad)


def _sc_edges(src, dst, xwp, aS, aD, N, E, P):
    epw = E // (_NCORE * _NSUB)
    nchunk = epw // _C
    rps = N // _NSUB
    ngrp = _C // 16
    zfull, zrem = rps // _C, rps % _C

    mesh = plsc.VectorSubcoreMesh(core_axis_name="c", subcore_axis_name="s")

    def body(src_h, dst_h, xwp_h, aS_h, aD_h, acc_o, dnm_o,
             acc_sp, dnm_sp, tS, tD,
             srcb0, srcb1, dstb0, dstb1, sidxb0, sidxb1,
             rows0, rows1, wbuf0, wbuf1, w0b0, w0b1,
             gsem0, gsem1, lsem0, lsem1):
        cc = lax.axis_index("c")
        ss = lax.axis_index("s")
        wid = cc * _NSUB + ss
        lanes = lax.iota(jnp.int32, 16)
        zcol = jnp.zeros((16,), jnp.int32)
        zvec = jnp.zeros((16,), jnp.float32)

        srcb = (srcb0, srcb1)
        dstb = (dstb0, dstb1)
        sidxb = (sidxb0, sidxb1)
        rows = (rows0, rows1)
        wbuf = (wbuf0, wbuf1)
        w0b = (w0b0, w0b1)
        gsem = (gsem0, gsem1)
        lsem = (lsem0, lsem1)

        def zb1(e, _):
            wbuf[1][e, pl.ds(0, 16)] = zvec
            return 0
        lax.fori_loop(0, _C, zb1, 0)

        def fire_io(ci, b):
            eb = wid * epw + ci * _C
            pltpu.async_copy(src_h.at[pl.ds(eb, _C)], srcb[b], lsem[b])
            pltpu.async_copy(dst_h.at[pl.ds(eb, _C)], dstb[b], lsem[b])

        def wait_io(b):
            pltpu.make_async_copy(src_h.at[pl.ds(0, _C)], srcb[b], lsem[b]).wait()
            pltpu.make_async_copy(dst_h.at[pl.ds(0, _C)], dstb[b], lsem[b]).wait()

        def fire_gather(b):
            pltpu.async_copy(xwp_h.at[sidxb[b]], rows[b], gsem[b])

        def wait_gather(b):
            pltpu.make_async_copy(xwp_h.at[sidxb[b]], rows[b], gsem[b]).wait()

        def passbody(p, _):
            def zr(e, _):
                for j in range(_FW // 16):
                    rows[0][e, pl.ds(j * 16, 16)] = zvec
                wbuf[0][e, pl.ds(0, 16)] = zvec
                return 0
            lax.fori_loop(0, _C, zr, 0)
            for z in range(zfull):
                pltpu.sync_copy(rows[0], acc_sp.at[pl.ds(ss * rps + z * _C, _C)])
                pltpu.sync_copy(wbuf[0], dnm_sp.at[pl.ds(ss * rps + z * _C, _C)])
            if zrem:
                pltpu.sync_copy(rows[0].at[pl.ds(0, zrem)],
                                acc_sp.at[pl.ds(ss * rps + zfull * _C, zrem)])
                pltpu.sync_copy(wbuf[0].at[pl.ds(0, zrem)],
                                dnm_sp.at[pl.ds(ss * rps + zfull * _C, zrem)])
            pltpu.sync_copy(aS_h.at[pl.ds(p * N, N)], tS)
            pltpu.sync_copy(aD_h.at[pl.ds(p * N, N)], tD)
            plsc.subcore_barrier()

            def do_grp(ci, b):
                def grp(g, _):
                    sl = pl.ds(g * 16, 16)
                    sv = srcb[b][sl]
                    dv = dstb[b][sl]
                    sidxb[b][sl] = sv + p * N
                    a1 = plsc.load_gather(tS, [sv])
                    a2 = plsc.load_gather(tD, [dv])
                    w = jnp.exp(_leaky(a1 + a2))
                    w0b[b][sl] = w
                    plsc.store_scatter(wbuf[b], [g * 16 + lanes, zcol], w)
                    return 0
                lax.fori_loop(0, ngrp, grp, 0)

            def do_mul(b):
                def mul(g, _):
                    base = g * 16
                    wv = w0b[b][pl.ds(base, 16)]
                    for i in range(16):
                        e = base + i
                        w = wv[i]
                        for j in range(_FW // 16):
                            sl = pl.ds(j * 16, 16)
                            rows[b][e, sl] = rows[b][e, sl] * w
                    return 0
                lax.fori_loop(0, ngrp, mul, 0)

            def step(ci, b, has_io2, has_next):
                nb = 1 - b
                wait_gather(b)
                do_mul(b)
                pltpu.sync_copy(rows[b], acc_sp.at[dstb[b]], add=True)
                pltpu.sync_copy(wbuf[b], dnm_sp.at[dstb[b]], add=True)
                if has_io2:
                    fire_io(ci + 2, b)
                if has_next:
                    wait_io(nb)
                    do_grp(ci + 1, nb)
                    fire_gather(nb)

            fire_io(0, 0)
            fire_io(1, 1)
            wait_io(0)
            do_grp(0, 0)
            fire_gather(0)
            npair = (nchunk - 3) // 2

            def pair(t, _):
                step(2 * t, 0, True, True)
                step(2 * t + 1, 1, True, True)
                return 0
            lax.fori_loop(0, npair, pair, 0)
            for ci in range(2 * npair, nchunk):
                step(ci, ci % 2, ci + 2 < nchunk, ci + 1 < nchunk)
            plsc.subcore_barrier()


            ob = (cc * P + p) * N + ss * rps
            for z in range(zfull):
                pltpu.sync_copy(acc_sp.at[pl.ds(ss * rps + z * _C, _C)],
                                acc_o.at[pl.ds(ob + z * _C, _C)])
                pltpu.sync_copy(dnm_sp.at[pl.ds(ss * rps + z * _C, _C)],
                                dnm_o.at[pl.ds(ob + z * _C, _C)])
            if zrem:
                pltpu.sync_copy(acc_sp.at[pl.ds(ss * rps + zfull * _C, zrem)],
                                acc_o.at[pl.ds(ob + zfull * _C, zrem)])
                pltpu.sync_copy(dnm_sp.at[pl.ds(ss * rps + zfull * _C, zrem)],
                                dnm_o.at[pl.ds(ob + zfull * _C, zrem)])
            plsc.subcore_barrier()
            return 0
        lax.fori_loop(0, P, passbody, 0)

    k = pl.kernel(
        body,
        out_type=(jax.ShapeDtypeStruct((_NCORE * P * N, _FW), jnp.float32),
                  jax.ShapeDtypeStruct((_NCORE * P * N, 16), jnp.float32)),
        mesh=mesh,
        compiler_params=pltpu.CompilerParams(needs_layout_passes=False,
                                             use_tc_tiling_on_sc=False),
        scratch_types=[
            pltpu.VMEM_SHARED((N, _FW), jnp.float32),
            pltpu.VMEM_SHARED((N, 16), jnp.float32),
            pltpu.VMEM((N,), jnp.float32),
            pltpu.VMEM((N,), jnp.float32),
            pltpu.VMEM((_C,), jnp.int32),
            pltpu.VMEM((_C,), jnp.int32),
            pltpu.VMEM((_C,), jnp.int32),
            pltpu.VMEM((_C,), jnp.int32),
            pltpu.VMEM((_C,), jnp.int32),
            pltpu.VMEM((_C,), jnp.int32),
            pltpu.VMEM((_C, _FW), jnp.float32),
            pltpu.VMEM((_C, _FW), jnp.float32),
            pltpu.VMEM((_C, 16), jnp.float32),
            pltpu.VMEM((_C, 16), jnp.float32),
            pltpu.VMEM((_C,), jnp.float32),
            pltpu.VMEM((_C,), jnp.float32),
            pltpu.SemaphoreType.DMA,
            pltpu.SemaphoreType.DMA,
            pltpu.SemaphoreType.DMA,
            pltpu.SemaphoreType.DMA,
        ],
    )
    return k(src, dst, xwp, aS, aD)


def _epi1_body(acc_ref, dnm_ref, asrc_ref, adst_ref, xw8_ref, b1_ref, w2_ref,
               as2_ref, ad2_ref, rmat_ref, xw2_ref, a2s_ref, a2d_ref):
    P = xw8_ref.shape[0]
    numer = jnp.concatenate(
        [acc_ref[0, p] + acc_ref[1, p] for p in range(P)], axis=1)
    dnm = jnp.concatenate(
        [dnm_ref[0, p][:, 0:1] + dnm_ref[1, p][:, 0:1] for p in range(P)],
        axis=1)
    xw = jnp.concatenate([xw8_ref[p] for p in range(P)], axis=1)
    selfw = jnp.exp(_leaky(asrc_ref[...] + adst_ref[...]))
    dnm = dnm + selfw + jnp.float32(1e-16)
    rmat = rmat_ref[...]
    numer = numer + xw * jnp.dot(selfw, rmat, preferred_element_type=jnp.float32)
    h = numer / jnp.dot(dnm, rmat, preferred_element_type=jnp.float32)
    h = h + b1_ref[...]
    h = jnp.where(h > 0, h, jnp.exp(jnp.minimum(h, 0.0)) - 1.0)
    xw2 = jnp.dot(h, w2_ref[...], preferred_element_type=jnp.float32)
    for q in range(xw2_ref.shape[0]):
        xw2_ref[q] = xw2[:, _FW * q:_FW * (q + 1)]
    a2s_ref[...] = jnp.sum(xw2 * as2_ref[...], axis=1, keepdims=True)
    a2d_ref[...] = jnp.sum(xw2 * ad2_ref[...], axis=1, keepdims=True)


def _epi1(acc, dnm, a_src, a_dst, xw8, bias1, W2, as2, ad2, rmat, N, K):
    P = xw8.shape[0]
    G = N // _RT
    Q = W2.shape[1] // _FW
    return pl.pallas_call(
        _epi1_body,
        grid=(G,),
        in_specs=[
            pl.BlockSpec((2, P, _RT, _FW), lambda i: (0, 0, i, 0)),
            pl.BlockSpec((2, P, _RT, 16), lambda i: (0, 0, i, 0)),
            pl.BlockSpec((_RT, K), lambda i: (i, 0)),
            pl.BlockSpec((_RT, K), lambda i: (i, 0)),
            pl.BlockSpec((P, _RT, _FW), lambda i: (0, i, 0)),
            pl.BlockSpec((1, bias1.shape[1]), lambda i: (0, 0)),
            pl.BlockSpec(W2.shape, lambda i: (0, 0)),
            pl.BlockSpec((1, 128), lambda i: (0, 0)),
            pl.BlockSpec((1, 128), lambda i: (0, 0)),
            pl.BlockSpec(rmat.shape, lambda i: (0, 0)),
        ],
        out_specs=[
            pl.BlockSpec((Q, _RT, _FW), lambda i: (0, i, 0)),
            pl.BlockSpec((_RT, 1), lambda i: (i, 0)),
            pl.BlockSpec((_RT, 1), lambda i: (i, 0)),
        ],
        out_shape=[
            jax.ShapeDtypeStruct((Q, N, _FW), jnp.float32),
            jax.ShapeDtypeStruct((N, 1), jnp.float32),
            jax.ShapeDtypeStruct((N, 1), jnp.float32),
        ],
    )(acc, dnm, a_src, a_dst, xw8, bias1, W2, as2, ad2, rmat)


def _epi2_body(acc_ref, dnm_ref, a2s_ref, a2d_ref, xw2_ref, b2_ref, o_ref):
    Q = xw2_ref.shape[0]
    selfw = jnp.exp(_leaky(a2s_ref[...] + a2d_ref[...]))
    numer = jnp.concatenate(
        [acc_ref[0, q] + acc_ref[1, q] for q in range(Q)], axis=1)
    xw2 = jnp.concatenate([xw2_ref[q] for q in range(Q)], axis=1)
    dnm = dnm_ref[0, 0][:, 0:1] + dnm_ref[1, 0][:, 0:1] + selfw + jnp.float32(1e-16)
    o = (numer + xw2 * selfw) / dnm + b2_ref[...]
    m = jnp.max(o, axis=1, keepdims=True)
    o_ref[...] = o - (m + jnp.log(jnp.sum(jnp.exp(o - m), axis=1, keepdims=True)))


def _epi2(acc2, dnm2, a2s, a2d, xw2h, bias2, N):
    G = N // _RT
    Q = xw2h.shape[0]
    return pl.pallas_call(
        _epi2_body,
        grid=(G,),
        in_specs=[
            pl.BlockSpec((2, Q, _RT, _FW), lambda i: (0, 0, i, 0)),
            pl.BlockSpec((2, Q, _RT, 16), lambda i: (0, 0, i, 0)),
            pl.BlockSpec((_RT, 1), lambda i: (i, 0)),
            pl.BlockSpec((_RT, 1), lambda i: (i, 0)),
            pl.BlockSpec((Q, _RT, _FW), lambda i: (0, i, 0)),
            pl.BlockSpec((1, 128), lambda i: (0, 0)),
        ],
        out_specs=pl.BlockSpec((_RT, 128), lambda i: (i, 0)),
        out_shape=jax.ShapeDtypeStruct((N, 128), jnp.float32),
    )(acc2, dnm2, a2s, a2d, xw2h, bias2)


def kernel(x, edge_index, W1, att_src1, att_dst1, bias1,
           W2, att_src2, att_dst2, bias2):
    N, D = x.shape
    E = edge_index.shape[1]
    K, HID = att_src1.shape[1], att_src1.shape[2]
    P1 = (K * HID) // _FW
    Q = W2.shape[1] // _FW

    src = edge_index[0]
    dst = edge_index[1]

    eye = jnp.eye(K, dtype=jnp.float32)
    As1 = (att_src1[0][:, :, None] * eye[:, None, :]).reshape(K * HID, K)
    Ad1 = (att_dst1[0][:, :, None] * eye[:, None, :]).reshape(K * HID, K)
    rmat = jnp.repeat(eye, HID, axis=1)

    xw8, a_src, a_dst = _prep1(x, W1, As1, Ad1, N, K, HID)
    aS1 = a_src.T.reshape(-1)
    aD1 = a_dst.T.reshape(-1)
    acc1, dnm1 = _sc_edges(src, dst, xw8.reshape(P1 * N, _FW), aS1, aD1,
                           N, E, P1)

    xw2h, a2s, a2d = _epi1(acc1.reshape(2, P1, N, _FW),
                           dnm1.reshape(2, P1, N, 16), a_src, a_dst, xw8,
                           bias1.reshape(1, -1), W2,
                           att_src2.reshape(1, 128), att_dst2.reshape(1, 128),
                           rmat, N, K)
    acc2, dnm2 = _sc_edges(src, dst, xw2h.reshape(Q * N, _FW),
                           jnp.tile(a2s.reshape(-1), Q),
                           jnp.tile(a2d.reshape(-1), Q), N, E, Q)

    return _epi2(acc2.reshape(2, Q, N, _FW), dnm2.reshape(2, Q, N, 16),
                 a2s, a2d, xw2h, bias2.reshape(1, 128), N)


# speedup vs baseline: 26.3103x; 1.8469x over previous
import jax
import jax.numpy as jnp
from jax import lax
from jax.experimental import pallas as pl
from jax.experimental.pallas import tpu as pltpu
from jax.experimental.pallas import tpu_sc as plsc

_FW = 64
_TW = 80
_RT = 400
_C = 80
_NCORE = 2
_NSUB = 16


def _leaky(x):
    return jnp.where(x >= 0, x, 0.2 * x)


def _prep1_body(x_ref, w1_ref, as_ref, ad_ref, xw8_ref, asrc_ref, adst_ref):
    xw = jnp.dot(x_ref[...], w1_ref[...], preferred_element_type=jnp.float32)
    asrc_ref[...] = jnp.dot(xw, as_ref[...], preferred_element_type=jnp.float32)
    adst_ref[...] = jnp.dot(xw, ad_ref[...], preferred_element_type=jnp.float32)
    r = xw.shape[0]
    one = jnp.ones((r, 1), jnp.float32)
    pad = jnp.zeros((r, _TW - _FW - 1), jnp.float32)
    for p in range(xw8_ref.shape[0]):
        xw8_ref[p] = jnp.concatenate(
            [xw[:, _FW * p:_FW * (p + 1)], one, pad], axis=1)


def _prep1(x, W1, As, Ad, N, K, HID):
    P = (K * HID) // _FW
    G = N // _RT
    return pl.pallas_call(
        _prep1_body,
        grid=(G,),
        in_specs=[
            pl.BlockSpec((_RT, x.shape[1]), lambda i: (i, 0)),
            pl.BlockSpec(W1.shape, lambda i: (0, 0)),
            pl.BlockSpec(As.shape, lambda i: (0, 0)),
            pl.BlockSpec(Ad.shape, lambda i: (0, 0)),
        ],
        out_specs=[
            pl.BlockSpec((P, _RT, _TW), lambda i: (0, i, 0)),
            pl.BlockSpec((_RT, K), lambda i: (i, 0)),
            pl.BlockSpec((_RT, K), lambda i: (i, 0)),
        ],
        out_shape=[
            jax.ShapeDtypeStruct((P, N, _TW), jnp.float32),
            jax.ShapeDtypeStruct((N, K), jnp.float32),
            jax.ShapeDtypeStruct((N, K), jnp.float32),
        ],
    )(x, W1, As, Ad)


def _sc_edges(src, dst, xwp, aS, aD, N, E, P):
    epw = E // (_NCORE * _NSUB)
    nchunk = epw // _C
    rps = N // _NSUB
    ngrp = _C // 16
    zfull, zrem = rps // _C, rps % _C

    mesh = plsc.VectorSubcoreMesh(core_axis_name="c", subcore_axis_name="s")

    def body(src_h, dst_h, xwp_h, aS_h, aD_h, acc_o,
             acc_sp, tS, tD,
             srcb0, srcb1, dstb0, dstb1, sidxb0, sidxb1, dsc0, dsc1,
             rows0, rows1, w0b0, w0b1,
             gsem0, gsem1, lsem0, lsem1, ssem0, ssem1):
        cc = lax.axis_index("c")
        ss = lax.axis_index("s")
        wid = cc * _NSUB + ss
        zvec = jnp.zeros((16,), jnp.float32)

        srcb = (srcb0, srcb1)
        dstb = (dstb0, dstb1)
        sidxb = (sidxb0, sidxb1)
        dsc = (dsc0, dsc1)
        rows = (rows0, rows1)
        w0b = (w0b0, w0b1)
        gsem = (gsem0, gsem1)
        lsem = (lsem0, lsem1)
        ssem = (ssem0, ssem1)

        def fire_io(ci, b):
            eb = wid * epw + ci * _C
            pltpu.async_copy(src_h.at[pl.ds(eb, _C)], srcb[b], lsem[b])
            pltpu.async_copy(dst_h.at[pl.ds(eb, _C)], dstb[b], lsem[b])

        def wait_io(b):
            pltpu.make_async_copy(src_h.at[pl.ds(0, _C)], srcb[b], lsem[b]).wait()
            pltpu.make_async_copy(dst_h.at[pl.ds(0, _C)], dstb[b], lsem[b]).wait()

        def fire_gather(b):
            pltpu.async_copy(xwp_h.at[sidxb[b]], rows[b], gsem[b])

        def wait_gather(b):
            pltpu.make_async_copy(xwp_h.at[sidxb[b]], rows[b], gsem[b]).wait()

        def fire_scatter(b):
            for g in range(ngrp):
                sl = pl.ds(g * 16, 16)
                dsc[b][sl] = dstb[b][sl]
            pltpu.async_copy(rows[b], acc_sp.at[dsc[b]], ssem[b], add=True)

        def wait_scatter(b):
            pltpu.make_async_copy(rows[b], acc_sp.at[dsc[b]], ssem[b]).wait()

        def passbody(p, _):
            def zr(e, _):
                for j in range(_TW // 16):
                    rows[0][e, pl.ds(j * 16, 16)] = zvec
                return 0
            lax.fori_loop(0, _C, zr, 0)
            for z in range(zfull):
                pltpu.sync_copy(rows[0], acc_sp.at[pl.ds(ss * rps + z * _C, _C)])
            if zrem:
                pltpu.sync_copy(rows[0].at[pl.ds(0, zrem)],
                                acc_sp.at[pl.ds(ss * rps + zfull * _C, zrem)])
            pltpu.sync_copy(aS_h.at[pl.ds(p * N, N)], tS)
            pltpu.sync_copy(aD_h.at[pl.ds(p * N, N)], tD)
            plsc.subcore_barrier()

            def do_grp(b):
                def grp(g, _):
                    sl = pl.ds(g * 16, 16)
                    sv = srcb[b][sl]
                    dv = dstb[b][sl]
                    sidxb[b][sl] = sv + p * N
                    a1 = plsc.load_gather(tS, [sv])
                    a2 = plsc.load_gather(tD, [dv])
                    w = jnp.exp(_leaky(a1 + a2))
                    w0b[b][sl] = w
                    return 0
                lax.fori_loop(0, ngrp, grp, 0)

            def do_mul(b):
                def mul(g, _):
                    base = g * 16
                    wv = w0b[b][pl.ds(base, 16)]
                    for i in range(16):
                        e = base + i
                        w = wv[i]
                        for j in range(_TW // 16):
                            sl = pl.ds(j * 16, 16)
                            rows[b][e, sl] = rows[b][e, sl] * w
                    return 0
                lax.fori_loop(0, ngrp, mul, 0)

            def step(ci, b, has_io2, has_next, has_prev_scat):
                nb = 1 - b
                wait_gather(b)
                do_mul(b)
                fire_scatter(b)
                if has_io2:
                    fire_io(ci + 2, b)
                if has_next:
                    wait_io(nb)
                    if has_prev_scat:
                        wait_scatter(nb)
                    do_grp(nb)
                    fire_gather(nb)

            fire_io(0, 0)
            fire_io(1, 1)
            wait_io(0)
            do_grp(0)
            fire_gather(0)
            step(0, 0, True, True, False)
            npair = (nchunk - 3) // 2

            def pair(t, _):
                step(2 * t + 1, 1, True, True, True)
                step(2 * t + 2, 0, True, True, True)
                return 0
            lax.fori_loop(0, npair, pair, 0)
            for ci in range(2 * npair + 1, nchunk):
                step(ci, ci % 2, ci + 2 < nchunk, ci + 1 < nchunk, True)
            wait_scatter(0)
            wait_scatter(1)
            plsc.subcore_barrier()

            ob = (cc * P + p) * N + ss * rps
            for z in range(zfull):
                pltpu.sync_copy(acc_sp.at[pl.ds(ss * rps + z * _C, _C)],
                                acc_o.at[pl.ds(ob + z * _C, _C)])
            if zrem:
                pltpu.sync_copy(acc_sp.at[pl.ds(ss * rps + zfull * _C, zrem)],
                                acc_o.at[pl.ds(ob + zfull * _C, zrem)])
            plsc.subcore_barrier()
            return 0
        lax.fori_loop(0, P, passbody, 0)

    k = pl.kernel(
        body,
        out_type=jax.ShapeDtypeStruct((_NCORE * P * N, _TW), jnp.float32),
        mesh=mesh,
        compiler_params=pltpu.CompilerParams(needs_layout_passes=False,
                                             use_tc_tiling_on_sc=False),
        scratch_types=[
            pltpu.VMEM_SHARED((N, _TW), jnp.float32),
            pltpu.VMEM((N,), jnp.float32),
            pltpu.VMEM((N,), jnp.float32),
            pltpu.VMEM((_C,), jnp.int32),
            pltpu.VMEM((_C,), jnp.int32),
            pltpu.VMEM((_C,), jnp.int32),
            pltpu.VMEM((_C,), jnp.int32),
            pltpu.VMEM((_C,), jnp.int32),
            pltpu.VMEM((_C,), jnp.int32),
            pltpu.VMEM((_C,), jnp.int32),
            pltpu.VMEM((_C,), jnp.int32),
            pltpu.VMEM((_C, _TW), jnp.float32),
            pltpu.VMEM((_C, _TW), jnp.float32),
            pltpu.VMEM((_C,), jnp.float32),
            pltpu.VMEM((_C,), jnp.float32),
            pltpu.SemaphoreType.DMA,
            pltpu.SemaphoreType.DMA,
            pltpu.SemaphoreType.DMA,
            pltpu.SemaphoreType.DMA,
            pltpu.SemaphoreType.DMA,
            pltpu.SemaphoreType.DMA,
        ],
    )
    return k(src, dst, xwp, aS, aD)


def _epi1_body(acc_ref, asrc_ref, adst_ref, xw8_ref, b1_ref, w2_ref,
               as2_ref, ad2_ref, rmat_ref, xw2_ref, a2s_ref, a2d_ref):
    P = xw8_ref.shape[0]
    numer = jnp.concatenate(
        [(acc_ref[0, p] + acc_ref[1, p])[:, :_FW] for p in range(P)], axis=1)
    dnm = jnp.concatenate(
        [acc_ref[0, p][:, _FW:_FW + 1] + acc_ref[1, p][:, _FW:_FW + 1]
         for p in range(P)], axis=1)
    xw = jnp.concatenate([xw8_ref[p][:, :_FW] for p in range(P)], axis=1)
    selfw = jnp.exp(_leaky(asrc_ref[...] + adst_ref[...]))
    dnm = dnm + selfw + jnp.float32(1e-16)
    rmat = rmat_ref[...]
    numer = numer + xw * jnp.dot(selfw, rmat, preferred_element_type=jnp.float32)
    h = numer / jnp.dot(dnm, rmat, preferred_element_type=jnp.float32)
    h = h + b1_ref[...]
    h = jnp.where(h > 0, h, jnp.exp(jnp.minimum(h, 0.0)) - 1.0)
    xw2 = jnp.dot(h, w2_ref[...], preferred_element_type=jnp.float32)
    r = xw2.shape[0]
    one = jnp.ones((r, 1), jnp.float32)
    pad = jnp.zeros((r, _TW - _FW - 1), jnp.float32)
    for q in range(xw2_ref.shape[0]):
        xw2_ref[q] = jnp.concatenate(
            [xw2[:, _FW * q:_FW * (q + 1)], one, pad], axis=1)
    a2s_ref[...] = jnp.sum(xw2 * as2_ref[...], axis=1, keepdims=True)
    a2d_ref[...] = jnp.sum(xw2 * ad2_ref[...], axis=1, keepdims=True)


def _epi1(acc, a_src, a_dst, xw8, bias1, W2, as2, ad2, rmat, N, K):
    P = xw8.shape[0]
    G = N // _RT
    Q = W2.shape[1] // _FW
    return pl.pallas_call(
        _epi1_body,
        grid=(G,),
        in_specs=[
            pl.BlockSpec((2, P, _RT, _TW), lambda i: (0, 0, i, 0)),
            pl.BlockSpec((_RT, K), lambda i: (i, 0)),
            pl.BlockSpec((_RT, K), lambda i: (i, 0)),
            pl.BlockSpec((P, _RT, _TW), lambda i: (0, i, 0)),
            pl.BlockSpec((1, bias1.shape[1]), lambda i: (0, 0)),
            pl.BlockSpec(W2.shape, lambda i: (0, 0)),
            pl.BlockSpec((1, 128), lambda i: (0, 0)),
            pl.BlockSpec((1, 128), lambda i: (0, 0)),
            pl.BlockSpec(rmat.shape, lambda i: (0, 0)),
        ],
        out_specs=[
            pl.BlockSpec((Q, _RT, _TW), lambda i: (0, i, 0)),
            pl.BlockSpec((_RT, 1), lambda i: (i, 0)),
            pl.BlockSpec((_RT, 1), lambda i: (i, 0)),
        ],
        out_shape=[
            jax.ShapeDtypeStruct((Q, N, _TW), jnp.float32),
            jax.ShapeDtypeStruct((N, 1), jnp.float32),
            jax.ShapeDtypeStruct((N, 1), jnp.float32),
        ],
    )(acc, a_src, a_dst, xw8, bias1, W2, as2, ad2, rmat)


def _epi2_body(acc_ref, a2s_ref, a2d_ref, xw2_ref, b2_ref, o_ref):
    Q = xw2_ref.shape[0]
    selfw = jnp.exp(_leaky(a2s_ref[...] + a2d_ref[...]))
    numer = jnp.concatenate(
        [(acc_ref[0, q] + acc_ref[1, q])[:, :_FW] for q in range(Q)], axis=1)
    xw2 = jnp.concatenate([xw2_ref[q][:, :_FW] for q in range(Q)], axis=1)
    dnm = (acc_ref[0, 0][:, _FW:_FW + 1] + acc_ref[1, 0][:, _FW:_FW + 1]
           + selfw + jnp.float32(1e-16))
    o = (numer + xw2 * selfw) / dnm + b2_ref[...]
    m = jnp.max(o, axis=1, keepdims=True)
    o_ref[...] = o - (m + jnp.log(jnp.sum(jnp.exp(o - m), axis=1, keepdims=True)))


def _epi2(acc2, a2s, a2d, xw2h, bias2, N):
    G = N // _RT
    Q = xw2h.shape[0]
    return pl.pallas_call(
        _epi2_body,
        grid=(G,),
        in_specs=[
            pl.BlockSpec((2, Q, _RT, _TW), lambda i: (0, 0, i, 0)),
            pl.BlockSpec((_RT, 1), lambda i: (i, 0)),
            pl.BlockSpec((_RT, 1), lambda i: (i, 0)),
            pl.BlockSpec((Q, _RT, _TW), lambda i: (0, i, 0)),
            pl.BlockSpec((1, 128), lambda i: (0, 0)),
        ],
        out_specs=pl.BlockSpec((_RT, 128), lambda i: (i, 0)),
        out_shape=jax.ShapeDtypeStruct((N, 128), jnp.float32),
    )(acc2, a2s, a2d, xw2h, bias2)


def kernel(x, edge_index, W1, att_src1, att_dst1, bias1,
           W2, att_src2, att_dst2, bias2):
    N, D = x.shape
    E = edge_index.shape[1]
    K, HID = att_src1.shape[1], att_src1.shape[2]
    P1 = (K * HID) // _FW
    Q = W2.shape[1] // _FW

    src = edge_index[0]
    dst = edge_index[1]

    eye = jnp.eye(K, dtype=jnp.float32)
    As1 = (att_src1[0][:, :, None] * eye[:, None, :]).reshape(K * HID, K)
    Ad1 = (att_dst1[0][:, :, None] * eye[:, None, :]).reshape(K * HID, K)
    rmat = jnp.repeat(eye, HID, axis=1)

    xw8, a_src, a_dst = _prep1(x, W1, As1, Ad1, N, K, HID)
    aS1 = a_src.T.reshape(-1)
    aD1 = a_dst.T.reshape(-1)
    acc1 = _sc_edges(src, dst, xw8.reshape(P1 * N, _TW), aS1, aD1, N, E, P1)

    xw2h, a2s, a2d = _epi1(acc1.reshape(2, P1, N, _TW), a_src, a_dst, xw8,
                           bias1.reshape(1, -1), W2,
                           att_src2.reshape(1, 128), att_dst2.reshape(1, 128),
                           rmat, N, K)
    acc2 = _sc_edges(src, dst, xw2h.reshape(Q * N, _TW),
                     jnp.tile(a2s.reshape(-1), Q),
                     jnp.tile(a2d.reshape(-1), Q), N, E, Q)

    return _epi2(acc2.reshape(2, Q, N, _TW), a2s, a2d, xw2h,
                 bias2.reshape(1, 128), N)


# speedup vs baseline: 33.2636x; 1.2643x over previous
import jax
import jax.numpy as jnp
from jax import lax
from jax.experimental import pallas as pl
from jax.experimental.pallas import tpu as pltpu
from jax.experimental.pallas import tpu_sc as plsc

_FW = 64
_TW = 80
_RT = 400
_C = 80
_NCORE = 2
_NSUB = 16


def _leaky(x):
    return jnp.where(x >= 0, x, 0.2 * x)


def _prep1_body(x_ref, w1_ref, as_ref, ad_ref, xw8_ref, asrc_ref, adst_ref):
    xw = jnp.dot(x_ref[...], w1_ref[...], preferred_element_type=jnp.float32)
    asrc_ref[...] = jnp.dot(xw, as_ref[...], preferred_element_type=jnp.float32)
    adst_ref[...] = jnp.dot(xw, ad_ref[...], preferred_element_type=jnp.float32)
    r = xw.shape[0]
    one = jnp.ones((r, 1), jnp.float32)
    pad = jnp.zeros((r, _TW - _FW - 1), jnp.float32)
    for p in range(xw8_ref.shape[0]):
        xw8_ref[p] = jnp.concatenate(
            [xw[:, _FW * p:_FW * (p + 1)], one, pad], axis=1)


def _prep1(x, W1, As, Ad, N, K, HID):
    P = (K * HID) // _FW
    G = N // _RT
    return pl.pallas_call(
        _prep1_body,
        grid=(G,),
        in_specs=[
            pl.BlockSpec((_RT, x.shape[1]), lambda i: (i, 0)),
            pl.BlockSpec(W1.shape, lambda i: (0, 0)),
            pl.BlockSpec(As.shape, lambda i: (0, 0)),
            pl.BlockSpec(Ad.shape, lambda i: (0, 0)),
        ],
        out_specs=[
            pl.BlockSpec((P, _RT, _TW), lambda i: (0, i, 0)),
            pl.BlockSpec((_RT, K), lambda i: (i, 0)),
            pl.BlockSpec((_RT, K), lambda i: (i, 0)),
        ],
        out_shape=[
            jax.ShapeDtypeStruct((P, N, _TW), jnp.float32),
            jax.ShapeDtypeStruct((N, K), jnp.float32),
            jax.ShapeDtypeStruct((N, K), jnp.float32),
        ],
    )(x, W1, As, Ad)


def _sc_edges(src, dst, xwp, aS, aD, N, E, P):
    epw = E // (_NCORE * _NSUB)
    nchunk = epw // _C
    rps = N // _NSUB
    ngrp = _C // 16
    zfull, zrem = rps // _C, rps % _C

    mesh = plsc.VectorSubcoreMesh(core_axis_name="c", subcore_axis_name="s")

    def body(src_h, dst_h, xwp_h, aS_h, aD_h, acc_o,
             acc_sp, tS, tD,
             srcb0, srcb1, dstb0, dstb1, sidxb0, sidxb1,
             dsc0, dsc1, dsc2, rows0, rows1, rows2, w0b0, w0b1,
             gsem0, gsem1, gsem2, lsem0, lsem1, ssem0, ssem1, ssem2):
        cc = lax.axis_index("c")
        ss = lax.axis_index("s")
        wid = cc * _NSUB + ss
        lanes = lax.iota(jnp.int32, 16)
        zvec = jnp.zeros((16,), jnp.float32)

        srcb = (srcb0, srcb1)
        dstb = (dstb0, dstb1)
        sidxb = (sidxb0, sidxb1)
        dsc = (dsc0, dsc1, dsc2)
        rows = (rows0, rows1, rows2)
        w0b = (w0b0, w0b1)
        gsem = (gsem0, gsem1, gsem2)
        lsem = (lsem0, lsem1)
        ssem = (ssem0, ssem1, ssem2)

        def fire_io(ci, b):
            eb = wid * epw + ci * _C
            pltpu.async_copy(src_h.at[pl.ds(eb, _C)], srcb[b], lsem[b])
            pltpu.async_copy(dst_h.at[pl.ds(eb, _C)], dstb[b], lsem[b])

        def wait_io(b):
            pltpu.make_async_copy(src_h.at[pl.ds(0, _C)], srcb[b], lsem[b]).wait()
            pltpu.make_async_copy(dst_h.at[pl.ds(0, _C)], dstb[b], lsem[b]).wait()

        def fire_gather(b2, b3):
            pltpu.async_copy(xwp_h.at[sidxb[b2]], rows[b3], gsem[b3])

        def wait_gather(b2, b3):
            pltpu.make_async_copy(xwp_h.at[sidxb[b2]], rows[b3], gsem[b3]).wait()

        def fire_scatter(b2, b3):
            for g in range(ngrp):
                sl = pl.ds(g * 16, 16)
                dsc[b3][sl] = dstb[b2][sl]
            pltpu.async_copy(rows[b3], acc_sp.at[dsc[b3]], ssem[b3], add=True)

        def wait_scatter(b3):
            pltpu.make_async_copy(rows[b3], acc_sp.at[dsc[b3]], ssem[b3]).wait()

        def passbody(p, _):
            def zr(e, _):
                for j in range(_TW // 16):
                    rows[0][e, pl.ds(j * 16, 16)] = zvec
                return 0
            lax.fori_loop(0, _C, zr, 0)
            for z in range(zfull):
                pltpu.sync_copy(rows[0], acc_sp.at[pl.ds(ss * rps + z * _C, _C)])
            if zrem:
                pltpu.sync_copy(rows[0].at[pl.ds(0, zrem)],
                                acc_sp.at[pl.ds(ss * rps + zfull * _C, zrem)])
            pltpu.sync_copy(aS_h.at[pl.ds(p * N, N)], tS)
            pltpu.sync_copy(aD_h.at[pl.ds(p * N, N)], tD)
            plsc.subcore_barrier()

            def do_grp(b):
                def grp(g, _):
                    sl = pl.ds(g * 16, 16)
                    sv = srcb[b][sl]
                    dv = dstb[b][sl]
                    sidxb[b][sl] = sv + p * N
                    a1 = plsc.load_gather(tS, [sv])
                    a2 = plsc.load_gather(tD, [dv])
                    w = jnp.exp(_leaky(a1 + a2))
                    w0b[b][sl] = w
                    return 0
                lax.fori_loop(0, ngrp, grp, 0)

            def do_mul(b2, b3):
                def mul(g, _):
                    base = g * 16
                    wv = w0b[b2][pl.ds(base, 16)]
                    for i in range(16):
                        e = base + i
                        w = wv[i]
                        for j in range(_FW // 16):
                            sl = pl.ds(j * 16, 16)
                            rows[b3][e, sl] = rows[b3][e, sl] * w
                        rows[b3][e, pl.ds(_FW, 16)] = jnp.where(
                            lanes == 0, w, jnp.float32(0.0))
                    return 0
                lax.fori_loop(0, ngrp, mul, 0)

            def step(ci, b2, b3, has_io2, has_next, has_scat_m2):
                nb2 = 1 - b2
                nb3 = (b3 + 1) % 3
                if has_next:
                    wait_io(nb2)
                    do_grp(nb2)
                if has_scat_m2:
                    wait_scatter(nb3)
                if has_next:
                    fire_gather(nb2, nb3)
                wait_gather(b2, b3)
                do_mul(b2, b3)
                fire_scatter(b2, b3)
                if has_io2:
                    fire_io(ci + 2, b2)

            fire_io(0, 0)
            fire_io(1, 1)
            wait_io(0)
            do_grp(0)
            fire_gather(0, 0)
            step(0, 0, 0, True, True, False)
            step(1, 1, 1, True, True, False)
            nsix = (nchunk - 5) // 6

            def six(t, _):
                ci = 6 * t
                step(ci + 2, 0, 2, True, True, True)
                step(ci + 3, 1, 0, True, True, True)
                step(ci + 4, 0, 1, True, True, True)
                step(ci + 5, 1, 2, True, True, True)
                step(ci + 6, 0, 0, True, True, True)
                step(ci + 7, 1, 1, True, True, True)
                return 0
            lax.fori_loop(0, nsix, six, 0)
            for ci in range(6 * nsix + 2, nchunk):
                step(ci, ci % 2, ci % 3, ci + 2 < nchunk, ci + 1 < nchunk, True)
            wait_scatter((nchunk - 2) % 3)
            wait_scatter((nchunk - 1) % 3)
            plsc.subcore_barrier()

            ob = (cc * P + p) * N + ss * rps
            for z in range(zfull):
                pltpu.sync_copy(acc_sp.at[pl.ds(ss * rps + z * _C, _C)],
                                acc_o.at[pl.ds(ob + z * _C, _C)])
            if zrem:
                pltpu.sync_copy(acc_sp.at[pl.ds(ss * rps + zfull * _C, zrem)],
                                acc_o.at[pl.ds(ob + zfull * _C, zrem)])
            plsc.subcore_barrier()
            return 0
        lax.fori_loop(0, P, passbody, 0)

    k = pl.kernel(
        body,
        out_type=jax.ShapeDtypeStruct((_NCORE * P * N, _TW), jnp.float32),
        mesh=mesh,
        compiler_params=pltpu.CompilerParams(needs_layout_passes=False,
                                             use_tc_tiling_on_sc=False),
        scratch_types=[
            pltpu.VMEM_SHARED((N, _TW), jnp.float32),
            pltpu.VMEM((N,), jnp.float32),
            pltpu.VMEM((N,), jnp.float32),
            pltpu.VMEM((_C,), jnp.int32),
            pltpu.VMEM((_C,), jnp.int32),
            pltpu.VMEM((_C,), jnp.int32),
            pltpu.VMEM((_C,), jnp.int32),
            pltpu.VMEM((_C,), jnp.int32),
            pltpu.VMEM((_C,), jnp.int32),
            pltpu.VMEM((_C,), jnp.int32),
            pltpu.VMEM((_C,), jnp.int32),
            pltpu.VMEM((_C,), jnp.int32),
            pltpu.VMEM((_C, _TW), jnp.float32),
            pltpu.VMEM((_C, _TW), jnp.float32),
            pltpu.VMEM((_C, _TW), jnp.float32),
            pltpu.VMEM((_C,), jnp.float32),
            pltpu.VMEM((_C,), jnp.float32),
            pltpu.SemaphoreType.DMA,
            pltpu.SemaphoreType.DMA,
            pltpu.SemaphoreType.DMA,
            pltpu.SemaphoreType.DMA,
            pltpu.SemaphoreType.DMA,
            pltpu.SemaphoreType.DMA,
            pltpu.SemaphoreType.DMA,
            pltpu.SemaphoreType.DMA,
        ],
    )
    return k(src, dst, xwp, aS, aD)


def _epi1_body(acc_ref, asrc_ref, adst_ref, xw8_ref, b1_ref, w2_ref,
               as2_ref, ad2_ref, rmat_ref, xw2_ref, a2s_ref, a2d_ref):
    P = xw8_ref.shape[0]
    numer = jnp.concatenate(
        [(acc_ref[0, p] + acc_ref[1, p])[:, :_FW] for p in range(P)], axis=1)
    dnm = jnp.concatenate(
        [acc_ref[0, p][:, _FW:_FW + 1] + acc_ref[1, p][:, _FW:_FW + 1]
         for p in range(P)], axis=1)
    xw = jnp.concatenate([xw8_ref[p][:, :_FW] for p in range(P)], axis=1)
    selfw = jnp.exp(_leaky(asrc_ref[...] + adst_ref[...]))
    dnm = dnm + selfw + jnp.float32(1e-16)
    rmat = rmat_ref[...]
    numer = numer + xw * jnp.dot(selfw, rmat, preferred_element_type=jnp.float32)
    h = numer / jnp.dot(dnm, rmat, preferred_element_type=jnp.float32)
    h = h + b1_ref[...]
    h = jnp.where(h > 0, h, jnp.exp(jnp.minimum(h, 0.0)) - 1.0)
    xw2 = jnp.dot(h, w2_ref[...], preferred_element_type=jnp.float32)
    r = xw2.shape[0]
    one = jnp.ones((r, 1), jnp.float32)
    pad = jnp.zeros((r, _TW - _FW - 1), jnp.float32)
    for q in range(xw2_ref.shape[0]):
        xw2_ref[q] = jnp.concatenate(
            [xw2[:, _FW * q:_FW * (q + 1)], one, pad], axis=1)
    a2s_ref[...] = jnp.sum(xw2 * as2_ref[...], axis=1, keepdims=True)
    a2d_ref[...] = jnp.sum(xw2 * ad2_ref[...], axis=1, keepdims=True)


def _epi1(acc, a_src, a_dst, xw8, bias1, W2, as2, ad2, rmat, N, K):
    P = xw8.shape[0]
    G = N // _RT
    Q = W2.shape[1] // _FW
    return pl.pallas_call(
        _epi1_body,
        grid=(G,),
        in_specs=[
            pl.BlockSpec((2, P, _RT, _TW), lambda i: (0, 0, i, 0)),
            pl.BlockSpec((_RT, K), lambda i: (i, 0)),
            pl.BlockSpec((_RT, K), lambda i: (i, 0)),
            pl.BlockSpec((P, _RT, _TW), lambda i: (0, i, 0)),
            pl.BlockSpec((1, bias1.shape[1]), lambda i: (0, 0)),
            pl.BlockSpec(W2.shape, lambda i: (0, 0)),
            pl.BlockSpec((1, 128), lambda i: (0, 0)),
            pl.BlockSpec((1, 128), lambda i: (0, 0)),
            pl.BlockSpec(rmat.shape, lambda i: (0, 0)),
        ],
        out_specs=[
            pl.BlockSpec((Q, _RT, _TW), lambda i: (0, i, 0)),
            pl.BlockSpec((_RT, 1), lambda i: (i, 0)),
            pl.BlockSpec((_RT, 1), lambda i: (i, 0)),
        ],
        out_shape=[
            jax.ShapeDtypeStruct((Q, N, _TW), jnp.float32),
            jax.ShapeDtypeStruct((N, 1), jnp.float32),
            jax.ShapeDtypeStruct((N, 1), jnp.float32),
        ],
    )(acc, a_src, a_dst, xw8, bias1, W2, as2, ad2, rmat)


def _epi2_body(acc_ref, a2s_ref, a2d_ref, xw2_ref, b2_ref, o_ref):
    Q = xw2_ref.shape[0]
    selfw = jnp.exp(_leaky(a2s_ref[...] + a2d_ref[...]))
    numer = jnp.concatenate(
        [(acc_ref[0, q] + acc_ref[1, q])[:, :_FW] for q in range(Q)], axis=1)
    xw2 = jnp.concatenate([xw2_ref[q][:, :_FW] for q in range(Q)], axis=1)
    dnm = (acc_ref[0, 0][:, _FW:_FW + 1] + acc_ref[1, 0][:, _FW:_FW + 1]
           + selfw + jnp.float32(1e-16))
    o = (numer + xw2 * selfw) / dnm + b2_ref[...]
    m = jnp.max(o, axis=1, keepdims=True)
    o_ref[...] = o - (m + jnp.log(jnp.sum(jnp.exp(o - m), axis=1, keepdims=True)))


def _epi2(acc2, a2s, a2d, xw2h, bias2, N):
    G = N // _RT
    Q = xw2h.shape[0]
    return pl.pallas_call(
        _epi2_body,
        grid=(G,),
        in_specs=[
            pl.BlockSpec((2, Q, _RT, _TW), lambda i: (0, 0, i, 0)),
            pl.BlockSpec((_RT, 1), lambda i: (i, 0)),
            pl.BlockSpec((_RT, 1), lambda i: (i, 0)),
            pl.BlockSpec((Q, _RT, _TW), lambda i: (0, i, 0)),
            pl.BlockSpec((1, 128), lambda i: (0, 0)),
        ],
        out_specs=pl.BlockSpec((_RT, 128), lambda i: (i, 0)),
        out_shape=jax.ShapeDtypeStruct((N, 128), jnp.float32),
    )(acc2, a2s, a2d, xw2h, bias2)


def kernel(x, edge_index, W1, att_src1, att_dst1, bias1,
           W2, att_src2, att_dst2, bias2):
    N, D = x.shape
    E = edge_index.shape[1]
    K, HID = att_src1.shape[1], att_src1.shape[2]
    P1 = (K * HID) // _FW
    Q = W2.shape[1] // _FW

    src = edge_index[0]
    dst = edge_index[1]

    eye = jnp.eye(K, dtype=jnp.float32)
    As1 = (att_src1[0][:, :, None] * eye[:, None, :]).reshape(K * HID, K)
    Ad1 = (att_dst1[0][:, :, None] * eye[:, None, :]).reshape(K * HID, K)
    rmat = jnp.repeat(eye, HID, axis=1)

    xw8, a_src, a_dst = _prep1(x, W1, As1, Ad1, N, K, HID)
    aS1 = a_src.T.reshape(-1)
    aD1 = a_dst.T.reshape(-1)
    acc1 = _sc_edges(src, dst, xw8.reshape(P1 * N, _TW), aS1, aD1, N, E, P1)

    xw2h, a2s, a2d = _epi1(acc1.reshape(2, P1, N, _TW), a_src, a_dst, xw8,
                           bias1.reshape(1, -1), W2,
                           att_src2.reshape(1, 128), att_dst2.reshape(1, 128),
                           rmat, N, K)
    acc2 = _sc_edges(src, dst, xw2h.reshape(Q * N, _TW),
                     jnp.tile(a2s.reshape(-1), Q),
                     jnp.tile(a2d.reshape(-1), Q), N, E, Q)

    return _epi2(acc2.reshape(2, Q, N, _TW), a2s, a2d, xw2h,
                 bias2.reshape(1, 128), N)
